# Initial kernel scaffold; baseline (speedup 1.0000x reference)
import jax
import jax.numpy as jnp
from jax.experimental import pallas as pl


def kernel(x, edge_index, edge_attr, gcn_Wrel, gcn_brel, gcn_Wroot, mlp_W, mlp_b, out_W, out_b):
    raise NotImplementedError("write your pallas kernel here")


# speedup vs baseline: 2.4298x; 2.4298x over previous
import functools

import jax
import jax.numpy as jnp
from jax import lax
from jax.experimental import pallas as pl
from jax.experimental.pallas import tpu as pltpu
from jax.experimental.pallas import tpu_sc as plsc

_NC = 2
_NS = 16
_NW = _NC * _NS
_EBLK = 128
_BN = 1000


def _pad_nodes(n):
    return -(-n // (_NS * 8)) * (_NS * 8)


@functools.lru_cache(maxsize=None)
def _make_edge_agg(n_nodes, wc, nblk):
    n_pad = _pad_nodes(n_nodes)
    rows_per_tile = n_pad // _NS
    nch = wc // 16
    mesh = plsc.VectorSubcoreMesh(core_axis_name="c", subcore_axis_name="s")

    @functools.partial(
        pl.kernel,
        out_type=jax.ShapeDtypeStruct((_NC, n_pad, wc), jnp.float32),
        mesh=mesh,
        scratch_types=[
            pltpu.VMEM((4, _EBLK), jnp.int32),
            pltpu.VMEM((_EBLK, wc), jnp.float32),
            pltpu.VMEM((1, wc), jnp.float32),
            pltpu.VMEM((16,), jnp.int32),
            pltpu.VMEM_SHARED((n_pad, wc), jnp.float32),
        ],
        compiler_params=pltpu.CompilerParams(use_tc_tiling_on_sc=False, needs_layout_passes=False),
    )
    def agg(m_hbm, eb_hbm, s0_hbm, s1_hbm, zero_hbm, out_hbm,
            eb_v, rows_v, stage_v, bnd_v, acc_sh):
        cid = lax.axis_index("c")
        sid = lax.axis_index("s")
        wid = cid * _NS + sid
        r0 = sid * rows_per_tile
        pltpu.sync_copy(zero_hbm.at[pl.ds(r0, rows_per_tile)],
                        acc_sh.at[pl.ds(r0, rows_per_tile)])
        pltpu.sync_copy(s0_hbm.at[wid], bnd_v)
        start = bnd_v[...][0]
        pltpu.sync_copy(s1_hbm.at[wid], bnd_v)
        end = bnd_v[...][0]
        b0 = start // _EBLK
        b1 = (end + _EBLK - 1) // _EBLK
        plsc.subcore_barrier()

        zero_acc = tuple(jnp.zeros((16,), jnp.float32) for _ in range(nch))

        def block(j, accs):
            pltpu.sync_copy(eb_hbm.at[j], eb_v)
            pltpu.sync_copy(m_hbm.at[eb_v.at[0]], rows_v)
            base = j * _EBLK

            def group(g, accs):
                dstv = eb_v[1, pl.ds(g * 16, 16)]
                attrv = plsc.bitcast(eb_v[2, pl.ds(g * 16, 16)], jnp.float32)
                flagv = eb_v[3, pl.ds(g * 16, 16)]
                gbase = base + g * 16
                for i in range(16):
                    pos = gbase + i
                    inr = (pos >= start) & (pos < end)
                    a = jnp.where(inr, attrv[i], 0.0)
                    accs = tuple(
                        accs[c] + rows_v[g * 16 + i, pl.ds(c * 16, 16)] * a
                        for c in range(nch))
                    emit = inr & (flagv[i] != 0)
                    ridx = dstv[i]

                    @pl.when(emit)
                    def _():
                        for c in range(nch):
                            stage_v[0, pl.ds(c * 16, 16)] = accs[c]
                        pltpu.sync_copy(stage_v, acc_sh.at[pl.ds(ridx, 1)])

                    accs = tuple(jnp.where(emit, 0.0, accs[c])
                                 for c in range(nch))
                return accs

            return lax.fori_loop(0, _EBLK // 16, group, accs)

        lax.fori_loop(b0, b1, block, zero_acc)
        plsc.subcore_barrier()
        pltpu.sync_copy(acc_sh.at[pl.ds(r0, rows_per_tile)],
                        out_hbm.at[cid, pl.ds(r0, rows_per_tile)])

    return agg


def _edge_aggregate(m, eb, s0, s1, zeros):
    fn = _make_edge_agg(m.shape[0], m.shape[1], eb.shape[0])
    return fn(m, eb, s0, s1, zeros)


def _matmul(a, w):
    n, k = a.shape
    cout = w.shape[1]

    def body(a_ref, w_ref, o_ref):
        o_ref[...] = jnp.dot(a_ref[...], w_ref[...],
                             preferred_element_type=jnp.float32)

    return pl.pallas_call(
        body,
        grid=(n // _BN,),
        in_specs=[pl.BlockSpec((_BN, k), lambda i: (i, 0)),
                  pl.BlockSpec((k, cout), lambda i: (0, 0))],
        out_specs=pl.BlockSpec((_BN, cout), lambda i: (i, 0)),
        out_shape=jax.ShapeDtypeStruct((n, cout), jnp.float32),
    )(a, w)


def _assemble_agg(partials, n):
    nc = len(partials)
    wcs = [p.shape[2] for p in partials]
    w = sum(wcs)

    def body(*refs):
        o_ref = refs[-1]
        off = 0
        for c in range(nc):
            o_ref[:, off:off + wcs[c]] = refs[c][0] + refs[c][1]
            off += wcs[c]

    return pl.pallas_call(
        body,
        grid=(n // _BN,),
        in_specs=[pl.BlockSpec((2, _BN, wc), lambda i: (0, i, 0)) for wc in wcs],
        out_specs=pl.BlockSpec((_BN, w), lambda i: (i, 0)),
        out_shape=jax.ShapeDtypeStruct((n, w), jnp.float32),
    )(*partials)


def _gcn_combine(partials, h, wroot, brel, wrel):
    n, cin = h.shape
    cout = wroot.shape[1]
    nc = len(partials)
    wcs = [p.shape[2] for p in partials]

    if nc > 2:
        agg = _assemble_agg(partials, n)
        d1 = _matmul(agg, wrel)
        d2 = _matmul(h, wroot)
        b2 = brel.reshape(1, cout)

        def ebody(d1_ref, d2_ref, b_ref, o_ref):
            o_ref[...] = jnp.maximum((d1_ref[...] + b_ref[...]) + d2_ref[...],
                                     0.0)

        return pl.pallas_call(
            ebody,
            grid=(n // _BN,),
            in_specs=[pl.BlockSpec((_BN, cout), lambda i: (i, 0)),
                      pl.BlockSpec((_BN, cout), lambda i: (i, 0)),
                      pl.BlockSpec((1, cout), lambda i: (0, 0))],
            out_specs=pl.BlockSpec((_BN, cout), lambda i: (i, 0)),
            out_shape=jax.ShapeDtypeStruct((n, cout), jnp.float32),
        )(d1, d2, b2)

    def body(*refs):
        ps = refs[:nc]
        h_ref, wroot_ref, wrel_ref, b_ref, o_ref = refs[nc:]
        aggs = [p[0] + p[1] for p in ps]
        agg = aggs[0] if nc == 1 else jnp.concatenate(aggs, axis=1)
        acc = jnp.dot(agg, wrel_ref[...],
                      preferred_element_type=jnp.float32) + b_ref[...]
        acc = acc + jnp.dot(h_ref[...], wroot_ref[...],
                            preferred_element_type=jnp.float32)
        o_ref[...] = jnp.maximum(acc, 0.0)

    in_specs = [pl.BlockSpec((2, _BN, w), lambda i: (0, i, 0)) for w in wcs]
    in_specs.append(pl.BlockSpec((_BN, cin), lambda i: (i, 0)))
    in_specs.append(pl.BlockSpec((cin, cout), lambda i: (0, 0)))
    in_specs.append(pl.BlockSpec(wrel.shape, lambda i: (0, 0)))
    in_specs.append(pl.BlockSpec((1, cout), lambda i: (0, 0)))
    args = list(partials) + [h, wroot, wrel, brel.reshape(1, cout)]
    return pl.pallas_call(
        body,
        grid=(n // _BN,),
        in_specs=in_specs,
        out_specs=pl.BlockSpec((_BN, cout), lambda i: (i, 0)),
        out_shape=jax.ShapeDtypeStruct((n, cout), jnp.float32),
    )(*args)


def _mlp_head(h, mlp_W, mlp_b, out_W, out_b):
    n = h.shape[0]
    nl = len(mlp_W)
    wo = jnp.zeros((out_W.shape[0], 128), jnp.float32).at[:, :1].set(out_W)
    bo = jnp.zeros((1, 128), jnp.float32).at[0, 0].set(out_b[0])

    def body(*refs):
        h_ref = refs[0]
        o_ref = refs[-1]
        a = h_ref[...]
        for i in range(nl):
            w_ref, b_ref = refs[1 + 2 * i], refs[2 + 2 * i]
            a = jnp.maximum(jnp.dot(a, w_ref[...],
                                    preferred_element_type=jnp.float32)
                            + b_ref[...], 0.0)
        o_ref[...] = jnp.dot(a, refs[-3][...],
                             preferred_element_type=jnp.float32) + refs[-2][...]

    in_specs = [pl.BlockSpec((_BN, h.shape[1]), lambda i: (i, 0))]
    args = [h]
    for w, b in zip(mlp_W, mlp_b):
        in_specs.append(pl.BlockSpec(w.shape, lambda i: (0, 0)))
        in_specs.append(pl.BlockSpec((1, w.shape[1]), lambda i: (0, 0)))
        args += [w, b.reshape(1, -1)]
    in_specs.append(pl.BlockSpec(wo.shape, lambda i: (0, 0)))
    in_specs.append(pl.BlockSpec(bo.shape, lambda i: (0, 0)))
    args += [wo, bo]
    res = pl.pallas_call(
        body,
        grid=(n // _BN,),
        in_specs=in_specs,
        out_specs=pl.BlockSpec((_BN, 128), lambda i: (i, 0)),
        out_shape=jax.ShapeDtypeStruct((n, 128), jnp.float32),
    )(*args)
    return res[:, :1]


def _prep_edges(src, dst, attr):
    e = dst.shape[0]
    e_pad = -(-e // _EBLK) * _EBLK
    pad = e_pad - e
    order = jnp.argsort(dst, stable=True)
    ds = dst[order]
    ss = src[order]
    aa = attr[order]
    if pad:
        ds = jnp.concatenate([ds, jnp.full((pad,), ds[-1], jnp.int32)])
        ss = jnp.concatenate([ss, jnp.zeros((pad,), jnp.int32)])
        aa = jnp.concatenate([aa, jnp.zeros((pad,), jnp.float32)])
    flags = jnp.concatenate([(ds[1:] != ds[:-1]).astype(jnp.int32),
                             jnp.ones((1,), jnp.int32)])
    nblk = e_pad // _EBLK
    eb = jnp.stack([ss.reshape(nblk, _EBLK),
                    ds.reshape(nblk, _EBLK),
                    lax.bitcast_convert_type(aa, jnp.int32).reshape(nblk, _EBLK),
                    flags.reshape(nblk, _EBLK)], axis=1)
    tgt = jnp.arange(_NW, dtype=jnp.int32) * (e // _NW)
    starts = jnp.searchsorted(ds[:e], ds[tgt], side="left").astype(jnp.int32)
    ends = jnp.concatenate([starts[1:], jnp.array([e_pad], jnp.int32)])
    s0 = jnp.tile(starts[:, None], (1, 16))
    s1 = jnp.tile(ends[:, None], (1, 16))
    return eb, s0, s1


def kernel(x, edge_index, edge_attr, gcn_Wrel, gcn_brel, gcn_Wroot,
           mlp_W, mlp_b, out_W, out_b):
    n = x.shape[0]
    eb, s0, s1 = _prep_edges(edge_index[0], edge_index[1], edge_attr)
    n_pad = _pad_nodes(n)
    zeros = {32: jnp.zeros((n_pad, 32), jnp.float32),
             128: jnp.zeros((n_pad, 128), jnp.float32)}

    h = x
    for wrel, brel, wroot in zip(gcn_Wrel, gcn_brel, gcn_Wroot):
        w = h.shape[1]
        wc = min(w, 128)
        partials = []
        for c in range(w // wc):
            mc = h[:, c * wc:(c + 1) * wc] if w > wc else h
            partials.append(_edge_aggregate(mc, eb, s0, s1, zeros[wc]))
        h = _gcn_combine(partials, h, wroot, brel, wrel)

    return _mlp_head(h, mlp_W, mlp_b, out_W, out_b)
